# baseline (speedup 1.0000x reference)
import functools

import jax
import jax.numpy as jnp
from jax import lax
from jax.experimental import pallas as pl
from jax.experimental.pallas import tpu as pltpu
from jax.experimental.pallas import tpu_sc as plsc

N = 1024
CHANNEL = 64
CLASS_NUM = 100000
ALPHA = 0.85
T = 1.0

CBLK = 12800
NBLK = 8


def _gather_rows_sc(bank_flat, label):
    info = plsc.get_sparse_core_info()
    num_workers = info.num_cores * info.num_subcores
    b_per_w = N // num_workers
    elems = b_per_w * CHANNEL
    mesh = plsc.VectorSubcoreMesh(core_axis_name="c", subcore_axis_name="s")

    @functools.partial(
        pl.kernel,
        mesh=mesh,
        out_type=jax.ShapeDtypeStruct((N * CHANNEL,), jnp.float32),
        scratch_types=[
            pltpu.VMEM((b_per_w,), jnp.int32),
            pltpu.VMEM((elems,), jnp.int32),
            pltpu.VMEM((elems,), jnp.float32),
            pltpu.SemaphoreType.DMA,
        ],
        compiler_params=pltpu.CompilerParams(use_tc_tiling_on_sc=False,
                                             needs_layout_passes=False),
    )
    def gather_kernel(label_hbm, bank_hbm, out_hbm, lbl_v, idx_v, rows_v,
                      sem):
        wid = lax.axis_index("s") * info.num_cores + lax.axis_index("c")
        base = wid * b_per_w
        pltpu.sync_copy(label_hbm.at[pl.ds(base, b_per_w)], lbl_v)
        lane = lax.iota(jnp.int32, 16)
        for h in range(b_per_w // 16):
            lbl16 = lbl_v[pl.ds(h * 16, 16)]
            pos_base = (lane + h * 16) * CHANNEL
            for c in range(CHANNEL):
                plsc.store_scatter(idx_v, [pos_base + c], lbl16 + c * CLASS_NUM)
        copies = []
        for k in range(elems // 128):
            copies.append(pltpu.async_copy(
                bank_hbm.at[idx_v.at[pl.ds(k * 128, 128)]],
                rows_v.at[pl.ds(k * 128, 128)], sem))
        for cp in copies:
            cp.wait()
        pltpu.sync_copy(rows_v, out_hbm.at[pl.ds(base * CHANNEL, elems)])

    return gather_kernel(label, bank_flat)


def _sweep_body(ft_ref, bankt_ref, s_ref):
    i = pl.program_id(0)
    ft = ft_ref[...]
    blk = bankt_ref[...]
    s_blk = lax.dot_general(ft, blk, (((0,), (0,)), ((), ())),
                            preferred_element_type=jnp.float32) / T
    e = jnp.exp(s_blk)

    @pl.when(i < NBLK - 1)
    def _full():
        part = jnp.sum(e, axis=1, keepdims=True)

        @pl.when(i == 0)
        def _init():
            s_ref[...] = part

        @pl.when(i > 0)
        def _accum():
            s_ref[...] = s_ref[...] + part

    @pl.when(i == NBLK - 1)
    def _tail():
        valid = lax.broadcasted_iota(jnp.int32, (N, CBLK), 1) < (
            CLASS_NUM - (NBLK - 1) * CBLK)
        part = jnp.sum(jnp.where(valid, e, 0.0), axis=1, keepdims=True)
        s_ref[...] = s_ref[...] + part


def _combine_body(ft_ref, old_ref, lrow_ref, lcol_ref, s_ref, out_ref):
    ft = ft_ref[...]
    old = old_ref[...]
    lcol = lcol_ref[...]
    p = lax.dot_general(ft, old, (((0,), (1,)), ((), ())),
                        preferred_element_type=jnp.float32) / T
    q = lax.dot_general(ft, ft, (((0,), (0,)), ((), ())),
                        preferred_element_type=jnp.float32) / T
    lrow = lrow_ref[...]
    row_i = lax.broadcasted_iota(jnp.int32, (N, N), 0)
    col_i = lax.broadcasted_iota(jnp.int32, (N, N), 1)
    later_same = jnp.logical_and(lcol == lrow, row_i > col_i)
    winner = jnp.logical_not(jnp.any(later_same, axis=0, keepdims=True))
    wmask = jnp.broadcast_to(winner, (N, N))
    newv = ALPHA * p + q
    cmax = jnp.max(jnp.where(wmask, newv, -1e30), axis=1, keepdims=True)
    m_fin = jnp.maximum(cmax, 0.0)
    corr = jnp.sum(
        jnp.where(wmask, jnp.exp(newv - m_fin) - jnp.exp(p - m_fin), 0.0),
        axis=1, keepdims=True)
    total = s_ref[...] * jnp.exp(-m_fin) + corr
    lse = m_fin + jnp.log(total)
    pdiag = jnp.sum(jnp.where(row_i == col_i, p, 0.0), axis=1, keepdims=True)
    same_win = jnp.logical_and(lcol == lrow, wmask)
    qsel = jnp.sum(jnp.where(same_win, q, 0.0), axis=1, keepdims=True)
    pos = ALPHA * pdiag + qsel
    out_ref[...] = jnp.mean(lse - pos, axis=(0, 1), keepdims=True)


def kernel(f_normed, bank, label):
    ft = f_normed.T
    bankt = bank.T
    old = _gather_rows_sc(bankt.reshape(-1), label).reshape(N, CHANNEL)
    lrow = label.reshape(1, N)
    lcol = label.reshape(N, 1)
    s_raw = pl.pallas_call(
        _sweep_body,
        grid=(NBLK,),
        in_specs=[
            pl.BlockSpec((CHANNEL, N), lambda i: (0, 0)),
            pl.BlockSpec((CHANNEL, CBLK), lambda i: (0, i)),
        ],
        out_specs=pl.BlockSpec((N, 1), lambda i: (0, 0)),
        out_shape=jax.ShapeDtypeStruct((N, 1), jnp.float32),
    )(ft, bankt)
    out = pl.pallas_call(
        _combine_body,
        in_specs=[
            pl.BlockSpec((CHANNEL, N), lambda: (0, 0)),
            pl.BlockSpec((N, CHANNEL), lambda: (0, 0)),
            pl.BlockSpec((1, N), lambda: (0, 0)),
            pl.BlockSpec((N, 1), lambda: (0, 0)),
            pl.BlockSpec((N, 1), lambda: (0, 0)),
        ],
        out_specs=pl.BlockSpec((1, 1), lambda: (0, 0)),
        out_shape=jax.ShapeDtypeStruct((1, 1), jnp.float32),
    )(ft, old, lrow, lcol, s_raw)
    return out[0, 0]
